# baseline (speedup 1.0000x reference)
import functools

import jax
import jax.numpy as jnp
from jax import lax
from jax.experimental import pallas as pl
from jax.experimental.pallas import tpu as pltpu
from jax.experimental.pallas import tpu_sc as plsc

_N = 100000
_C = 80
_FLAT = _N * _C

_NW = 32
_BPW = 3200
_NPAD = _NW * _BPW
_NCHUNK = _BPW // 128

_G = 25
_S = _FLAT // _G // 128
_CB = _NPAD // _G // 128


def _sc_gather_body(pred_hbm, lab_hbm, out_hbm, lab_v, idx_v, rows_v, sem):
    wid = lax.axis_index("s") * 2 + lax.axis_index("c")
    base = wid * _BPW
    pltpu.sync_copy(lab_hbm.at[pl.ds(base, _BPW)], lab_v)
    lane = lax.iota(jnp.int32, 16)
    for c in range(_NCHUNK):
        def body(j, _, c=c):
            t = c * 8 + j
            lab = lab_v[pl.ds(t * 16, 16)]
            labc = jnp.clip(lab, 0, _C - 1)
            row = base + t * 16 + lane
            idx = jnp.minimum(row * _C + labc, _FLAT - 1)
            idx_v[c, pl.ds(j * 16, 16)] = idx
            return 0
        lax.fori_loop(0, 8, body, 0)
    copies = [
        pltpu.async_copy(pred_hbm.at[idx_v.at[c]], rows_v.at[c], sem)
        for c in range(_NCHUNK)
    ]
    for cp in copies:
        cp.wait()
    pltpu.sync_copy(rows_v, out_hbm.at[wid])


_sc_gather = pl.kernel(
    _sc_gather_body,
    out_type=jax.ShapeDtypeStruct((_NW, _NCHUNK, 128), jnp.float32),
    mesh=plsc.VectorSubcoreMesh(core_axis_name="c", subcore_axis_name="s"),
    scratch_types=[
        pltpu.VMEM((_BPW,), jnp.int32),
        pltpu.VMEM((_NCHUNK, 128), jnp.int32),
        pltpu.VMEM((_NCHUNK, 128), jnp.float32),
        pltpu.SemaphoreType.DMA,
    ],
)


def _tc_body(pred_ref, pa_ref, lab_ref, s_ref, w_ref, out_ref):
    p = pred_ref[0]
    log1mp = jnp.maximum(jnp.log(1.0 - p), -100.0)
    neg = log1mp * (p * p * -0.75)
    dense = jnp.sum(neg)

    pa = pa_ref[0]
    lab = lab_ref[0]
    s = s_ref[0]
    w = w_ref[0]
    pos_mask = (lab >= 0) & (lab < _C)
    lp = jnp.maximum(jnp.log(pa), -100.0)
    l1p = jnp.maximum(jnp.log(1.0 - pa), -100.0)
    pos_val = -(s * lp + (1.0 - s) * l1p) * w
    neg_at = l1p * (pa * pa * -0.75)
    corr = jnp.sum(jnp.where(pos_mask, pos_val - neg_at, 0.0))

    @pl.when(pl.program_id(0) == 0)
    def _init():
        out_ref[0, 0] = 0.0

    out_ref[0, 0] += dense + corr


def kernel(pred, label, score, weight):
    pred_flat = pred.reshape(_FLAT)
    lab_pad = jnp.pad(label, (0, _NPAD - _N), constant_values=_C)

    pred_at = _sc_gather(pred_flat, lab_pad)

    pred3 = pred_flat.reshape(_G, _S, 128)
    pa3 = pred_at.reshape(_G, _CB, 128)
    lab3 = lab_pad.reshape(_G, _CB, 128)
    s3 = jnp.pad(score, (0, _NPAD - _N)).reshape(_G, _CB, 128)
    w3 = jnp.pad(weight, (0, _NPAD - _N)).reshape(_G, _CB, 128)

    out = pl.pallas_call(
        _tc_body,
        grid=(_G,),
        in_specs=[
            pl.BlockSpec((1, _S, 128), lambda i: (i, 0, 0)),
            pl.BlockSpec((1, _CB, 128), lambda i: (i, 0, 0)),
            pl.BlockSpec((1, _CB, 128), lambda i: (i, 0, 0)),
            pl.BlockSpec((1, _CB, 128), lambda i: (i, 0, 0)),
            pl.BlockSpec((1, _CB, 128), lambda i: (i, 0, 0)),
        ],
        out_specs=pl.BlockSpec((1, 1), lambda i: (0, 0), memory_space=pltpu.SMEM),
        out_shape=jax.ShapeDtypeStruct((1, 1), jnp.float32),
    )(pred3, pa3, lab3, s3, w3)
    return out[0, 0] / _N


# speedup vs baseline: 6.3179x; 6.3179x over previous
import jax
import jax.numpy as jnp
from jax.experimental import pallas as pl
from jax.experimental.pallas import tpu as pltpu

_N = 100000
_C = 80
_BLKL = 2048
_GRID = -(-_N // _BLKL)


def _tc_body(predT_ref, lab_ref, s_ref, w_ref, out_ref):
    i = pl.program_id(0)
    col = jax.lax.broadcasted_iota(jnp.int32, (1, _BLKL), 1) + i * _BLKL
    valid = col < _N
    p = jnp.where(valid, predT_ref[...], 0.0)
    log1mp = jnp.maximum(jnp.log(1.0 - p), -100.0)
    neg = log1mp * (p * p * -0.75)

    lab = lab_ref[...].reshape(1, _BLKL)
    labc = jnp.clip(lab, 0, _C - 1)
    onehot = jax.lax.broadcasted_iota(jnp.int32, (_C, _BLKL), 0) == labc
    p_at = jnp.sum(jnp.where(onehot, p, 0.0), axis=0, keepdims=True)
    neg_at = jnp.sum(jnp.where(onehot, neg, 0.0), axis=0, keepdims=True)

    s = s_ref[...].reshape(1, _BLKL)
    w = w_ref[...].reshape(1, _BLKL)
    pos_mask = (lab >= 0) & (lab < _C) & valid
    lp = jnp.maximum(jnp.log(p_at), -100.0)
    l1p = jnp.maximum(jnp.log(1.0 - p_at), -100.0)
    pos_val = -(s * lp + (1.0 - s) * l1p) * w
    corr = jnp.where(pos_mask, pos_val - neg_at, 0.0)

    total = jnp.sum(neg) + jnp.sum(corr)

    @pl.when(i == 0)
    def _init():
        out_ref[0, 0] = 0.0

    out_ref[0, 0] += total


def kernel(pred, label, score, weight):
    out = pl.pallas_call(
        _tc_body,
        grid=(_GRID,),
        in_specs=[
            pl.BlockSpec((_C, _BLKL), lambda i: (0, i)),
            pl.BlockSpec((_BLKL,), lambda i: (i,)),
            pl.BlockSpec((_BLKL,), lambda i: (i,)),
            pl.BlockSpec((_BLKL,), lambda i: (i,)),
        ],
        out_specs=pl.BlockSpec((1, 1), lambda i: (0, 0), memory_space=pltpu.SMEM),
        out_shape=jax.ShapeDtypeStruct((1, 1), jnp.float32),
    )(pred.T, label, score, weight)
    return out[0, 0] / _N


# speedup vs baseline: 6.6224x; 1.0482x over previous
import jax
import jax.numpy as jnp
from jax.experimental import pallas as pl
from jax.experimental.pallas import tpu as pltpu

_N = 100000
_C = 80
_BLKL = 2048
_GRID = -(-_N // _BLKL)


def _tc_body(predT_ref, lab_ref, s_ref, w_ref, out_ref):
    i = pl.program_id(0)
    col = jax.lax.broadcasted_iota(jnp.int32, (1, _BLKL), 1) + i * _BLKL
    valid = col < _N
    p = jnp.where(valid, predT_ref[...], 0.0)
    log1mp = jnp.maximum(jnp.log(1.0 - p), -100.0)
    neg = log1mp * (p * p * -0.75)

    lab = lab_ref[...].reshape(1, _BLKL)
    labc = jnp.clip(lab, 0, _C - 1)
    onehot = jax.lax.broadcasted_iota(jnp.int32, (_C, _BLKL), 0) == labc
    p_at = jnp.sum(jnp.where(onehot, p, 0.0), axis=0, keepdims=True)

    s = s_ref[...].reshape(1, _BLKL)
    w = w_ref[...].reshape(1, _BLKL)
    pos_mask = (lab >= 0) & (lab < _C) & valid
    lp = jnp.maximum(jnp.log(p_at), -100.0)
    l1p = jnp.maximum(jnp.log(1.0 - p_at), -100.0)
    pos_val = -(s * lp + (1.0 - s) * l1p) * w
    neg_at = l1p * (p_at * p_at * -0.75)
    corr = jnp.where(pos_mask, pos_val - neg_at, 0.0)

    total = jnp.sum(neg) + jnp.sum(corr)

    @pl.when(i == 0)
    def _init():
        out_ref[0, 0] = 0.0

    out_ref[0, 0] += total


def kernel(pred, label, score, weight):
    out = pl.pallas_call(
        _tc_body,
        grid=(_GRID,),
        in_specs=[
            pl.BlockSpec((_C, _BLKL), lambda i: (0, i)),
            pl.BlockSpec((_BLKL,), lambda i: (i,)),
            pl.BlockSpec((_BLKL,), lambda i: (i,)),
            pl.BlockSpec((_BLKL,), lambda i: (i,)),
        ],
        out_specs=pl.BlockSpec((1, 1), lambda i: (0, 0), memory_space=pltpu.SMEM),
        out_shape=jax.ShapeDtypeStruct((1, 1), jnp.float32),
    )(pred.T, label, score, weight)
    return out[0, 0] / _N


# speedup vs baseline: 9.4436x; 1.4260x over previous
import jax
import jax.numpy as jnp
from jax.experimental import pallas as pl
from jax.experimental.pallas import tpu as pltpu

_N = 100000
_C = 80
_BLKL = 4096
_GRID = -(-_N // _BLKL)


def _tc_body(predT_ref, lab_ref, s_ref, w_ref, out_ref):
    i = pl.program_id(0)
    col = jax.lax.broadcasted_iota(jnp.int32, (1, _BLKL), 1) + i * _BLKL
    valid = col < _N
    p = jnp.where(valid, predT_ref[...], 0.0)
    log1mp = jnp.maximum(jnp.log(1.0 - p), -100.0)
    neg = log1mp * (p * p * -0.75)

    lab = lab_ref[...].reshape(1, _BLKL)
    labc = jnp.clip(lab, 0, _C - 1)
    onehot = jax.lax.broadcasted_iota(jnp.int32, (_C, _BLKL), 0) == labc
    p_at = jnp.sum(jnp.where(onehot, p, 0.0), axis=0, keepdims=True)

    s = s_ref[...].reshape(1, _BLKL)
    w = w_ref[...].reshape(1, _BLKL)
    pos_mask = (lab >= 0) & (lab < _C) & valid
    lp = jnp.maximum(jnp.log(p_at), -100.0)
    l1p = jnp.maximum(jnp.log(1.0 - p_at), -100.0)
    pos_val = -(s * lp + (1.0 - s) * l1p) * w
    neg_at = l1p * (p_at * p_at * -0.75)
    corr = jnp.where(pos_mask, pos_val - neg_at, 0.0)

    total = jnp.sum(neg) + jnp.sum(corr)

    @pl.when(i == 0)
    def _init():
        out_ref[0, 0] = 0.0

    out_ref[0, 0] += total


def kernel(pred, label, score, weight):
    out = pl.pallas_call(
        _tc_body,
        grid=(_GRID,),
        in_specs=[
            pl.BlockSpec((_C, _BLKL), lambda i: (0, i)),
            pl.BlockSpec((_BLKL,), lambda i: (i,)),
            pl.BlockSpec((_BLKL,), lambda i: (i,)),
            pl.BlockSpec((_BLKL,), lambda i: (i,)),
        ],
        out_specs=pl.BlockSpec((1, 1), lambda i: (0, 0), memory_space=pltpu.SMEM),
        out_shape=jax.ShapeDtypeStruct((1, 1), jnp.float32),
    )(pred.T, label, score, weight)
    return out[0, 0] / _N


# speedup vs baseline: 10.9481x; 1.1593x over previous
import jax
import jax.numpy as jnp
from jax.experimental import pallas as pl
from jax.experimental.pallas import tpu as pltpu

_N = 100000
_C = 80
_BLKL = 8192
_GRID = -(-_N // _BLKL)


def _tc_body(predT_ref, lab_ref, s_ref, w_ref, out_ref):
    i = pl.program_id(0)
    col = jax.lax.broadcasted_iota(jnp.int32, (1, _BLKL), 1) + i * _BLKL
    valid = col < _N
    p = jnp.where(valid, predT_ref[...], 0.0)
    log1mp = jnp.maximum(jnp.log(1.0 - p), -100.0)
    neg = log1mp * (p * p * -0.75)

    lab = lab_ref[...].reshape(1, _BLKL)
    labc = jnp.clip(lab, 0, _C - 1)
    onehot = jax.lax.broadcasted_iota(jnp.int32, (_C, _BLKL), 0) == labc
    p_at = jnp.sum(jnp.where(onehot, p, 0.0), axis=0, keepdims=True)

    s = s_ref[...].reshape(1, _BLKL)
    w = w_ref[...].reshape(1, _BLKL)
    pos_mask = (lab >= 0) & (lab < _C) & valid
    lp = jnp.maximum(jnp.log(p_at), -100.0)
    l1p = jnp.maximum(jnp.log(1.0 - p_at), -100.0)
    pos_val = -(s * lp + (1.0 - s) * l1p) * w
    neg_at = l1p * (p_at * p_at * -0.75)
    corr = jnp.where(pos_mask, pos_val - neg_at, 0.0)

    total = jnp.sum(neg) + jnp.sum(corr)

    @pl.when(i == 0)
    def _init():
        out_ref[0, 0] = 0.0

    out_ref[0, 0] += total


def kernel(pred, label, score, weight):
    out = pl.pallas_call(
        _tc_body,
        grid=(_GRID,),
        in_specs=[
            pl.BlockSpec((_C, _BLKL), lambda i: (0, i)),
            pl.BlockSpec((_BLKL,), lambda i: (i,)),
            pl.BlockSpec((_BLKL,), lambda i: (i,)),
            pl.BlockSpec((_BLKL,), lambda i: (i,)),
        ],
        out_specs=pl.BlockSpec((1, 1), lambda i: (0, 0), memory_space=pltpu.SMEM),
        out_shape=jax.ShapeDtypeStruct((1, 1), jnp.float32),
    )(pred.T, label, score, weight)
    return out[0, 0] / _N
